# Initial kernel scaffold; baseline (speedup 1.0000x reference)
import jax
import jax.numpy as jnp
from jax.experimental import pallas as pl


def kernel(x, edge_index, drpt, W1, b1, W2, b2):
    raise NotImplementedError("write your pallas kernel here")


# speedup vs baseline: 13.9667x; 13.9667x over previous
import functools

import jax
import jax.numpy as jnp
from jax import lax
from jax.experimental import pallas as pl
from jax.experimental.pallas import tpu as pltpu
from jax.experimental.pallas import tpu_sc as plsc

_N = 10000
_NP = 10240
_E = 320000
_D = 128
_NC, _NS = 2, 16
_NW = _NC * _NS
_EPT = _E // _NW
_K = 80
_NCHUNK = _EPT // _K
_RPT = _NP // _NS
_RB = 2000

@functools.cache
def _build_deg_kernel():
    mesh = plsc.VectorSubcoreMesh(
        core_axis_name="c", subcore_axis_name="s", num_cores=_NC, num_subcores=_NS
    )
    return pl.kernel(
        _deg_body,
        out_type=jax.ShapeDtypeStruct((_NC, _NP), jnp.float32),
        mesh=mesh,
        compiler_params=pltpu.CompilerParams(needs_layout_passes=False),
        scratch_types=[
            pltpu.VMEM((_EPT,), jnp.int32),
            pltpu.VMEM((_NP,), jnp.float32),
            pltpu.VMEM((_RPT,), jnp.float32),
            pltpu.VMEM((_RPT,), jnp.float32),
            pltpu.VMEM_SHARED((_NS, _NP), jnp.float32),
        ],
    )


def _deg_body(dst_hbm, out_hbm, dbuf, hist, racc, rtmp, sp):
    c = lax.axis_index("c")
    s = lax.axis_index("s")
    wid = s * _NC + c

    @pl.loop(0, _NP // 16)
    def _(i):
        hist[pl.ds(i * 16, 16)] = jnp.zeros((16,), jnp.float32)

    pltpu.sync_copy(dst_hbm.at[pl.ds(wid * _EPT, _EPT)], dbuf)
    ones = jnp.full((16,), 1.0, jnp.float32)

    @pl.loop(0, _EPT // 16)
    def _(j):
        idx = dbuf[pl.ds(j * 16, 16)]
        plsc.addupdate_scatter(hist, [idx], ones)

    pltpu.sync_copy(hist, sp.at[s])
    plsc.subcore_barrier()

    base = s * _RPT
    pltpu.sync_copy(sp.at[0, pl.ds(base, _RPT)], racc)
    for k in range(1, _NS):
        pltpu.sync_copy(sp.at[k, pl.ds(base, _RPT)], rtmp)

        @pl.loop(0, _RPT // 16)
        def _(i):
            sl = pl.ds(i * 16, 16)
            racc[sl] = racc[sl] + rtmp[sl]

    pltpu.sync_copy(racc, out_hbm.at[c, pl.ds(base, _RPT)])


@functools.cache
def _build_agg_kernel():
    mesh = plsc.VectorSubcoreMesh(
        core_axis_name="c", subcore_axis_name="s", num_cores=_NC, num_subcores=_NS
    )
    return pl.kernel(
        _agg_body,
        out_type=jax.ShapeDtypeStruct((_NC, _NP, _D), jnp.float32),
        mesh=mesh,
        compiler_params=pltpu.CompilerParams(needs_layout_passes=False),
        scratch_types=[
            pltpu.VMEM((_K,), jnp.int32),
            pltpu.VMEM((_K,), jnp.int32),
            pltpu.VMEM((_K, _D), jnp.float32),
            pltpu.VMEM((32, _D), jnp.float32),
            pltpu.VMEM_SHARED((_NP, _D), jnp.float32),
            pltpu.SemaphoreType.DMA,
        ],
    )


def _agg_body(g_hbm, src_hbm, dst_hbm, out_hbm, sidx, didx, rows, zbuf, acc, sem):
    c = lax.axis_index("c")
    s = lax.axis_index("s")
    wid = s * _NC + c

    for r in range(32):
        for q in range(_D // 16):
            zbuf[r, pl.ds(q * 16, 16)] = jnp.zeros((16,), jnp.float32)
    base = s * _RPT

    @pl.loop(0, _RPT // 32)
    def _(j):
        pltpu.sync_copy(zbuf, acc.at[pl.ds(base + j * 32, 32)])

    plsc.subcore_barrier()

    ebase = wid * _EPT

    @pl.loop(0, _NCHUNK)
    def _(j):
        off = ebase + j * _K
        pltpu.sync_copy(src_hbm.at[pl.ds(off, _K)], sidx)
        pltpu.sync_copy(dst_hbm.at[pl.ds(off, _K)], didx)
        pltpu.async_copy(g_hbm.at[sidx], rows, sem).wait()
        pltpu.sync_copy(rows, acc.at[didx], add=True)

    plsc.subcore_barrier()
    pltpu.sync_copy(acc.at[pl.ds(base, _RPT)], out_hbm.at[c, pl.ds(base, _RPT)])


def _tc1_body(x_ref, w_ref, d0_ref, d1_ref, g_ref, dinv_ref):
    dinv = lax.rsqrt(d0_ref[:] + d1_ref[:] + 1.0)
    g_ref[:] = jnp.dot(x_ref[:], w_ref[:], preferred_element_type=jnp.float32) * dinv
    dinv_ref[:] = dinv


def _tc2_body(p0_ref, p1_ref, g1_ref, dinv_ref, b1_ref, w2_ref, g2_ref):
    dinv = dinv_ref[:]
    h = jnp.maximum((p0_ref[:] + p1_ref[:] + g1_ref[:]) * dinv + b1_ref[:], 0.0)
    g2_ref[:] = jnp.dot(h, w2_ref[:], preferred_element_type=jnp.float32) * dinv


def _tc3_body(p0_ref, p1_ref, g2_ref, dinv_ref, b2_ref, o_ref):
    o_ref[:] = (p0_ref[:] + p1_ref[:] + g2_ref[:]) * dinv_ref[:] + b2_ref[:]


_row_spec = pl.BlockSpec((_RB, _D), lambda i: (i, 0))
_col_spec = pl.BlockSpec((_RB, 1), lambda i: (i, 0))
_w_spec = pl.BlockSpec((_D, _D), lambda i: (0, 0))
_b_spec = pl.BlockSpec((1, _D), lambda i: (0, 0))
_GRID = _N // _RB

_tc1 = pl.pallas_call(
    _tc1_body,
    grid=(_GRID,),
    in_specs=[_row_spec, _w_spec, _col_spec, _col_spec],
    out_specs=[_row_spec, _col_spec],
    out_shape=[
        jax.ShapeDtypeStruct((_N, _D), jnp.float32),
        jax.ShapeDtypeStruct((_N, 1), jnp.float32),
    ],
)

_tc2 = pl.pallas_call(
    _tc2_body,
    grid=(_GRID,),
    in_specs=[_row_spec, _row_spec, _row_spec, _col_spec, _b_spec, _w_spec],
    out_specs=_row_spec,
    out_shape=jax.ShapeDtypeStruct((_N, _D), jnp.float32),
)

_tc3 = pl.pallas_call(
    _tc3_body,
    grid=(_GRID,),
    in_specs=[_row_spec, _row_spec, _row_spec, _col_spec, _b_spec],
    out_specs=_row_spec,
    out_shape=jax.ShapeDtypeStruct((_N, _D), jnp.float32),
)


def kernel(x, edge_index, drpt, W1, b1, W2, b2):
    del drpt
    src = edge_index[0]
    dst = edge_index[1]

    deg_kernel = _build_deg_kernel()
    agg_kernel = _build_agg_kernel()

    deg2 = deg_kernel(dst)
    d0 = deg2[0, :_N, None]
    d1 = deg2[1, :_N, None]

    g1, dinv = _tc1(x, W1, d0, d1)
    p1 = agg_kernel(g1, src, dst)
    g2 = _tc2(p1[0, :_N], p1[1, :_N], g1, dinv, b1[None], W2)
    p2 = agg_kernel(g2, src, dst)
    out = _tc3(p2[0, :_N], p2[1, :_N], g2, dinv, b2[None])
    return out
